# Initial kernel scaffold; baseline (speedup 1.0000x reference)
import functools

import jax
import jax.numpy as jnp
from jax import lax
from jax.experimental import pallas as pl
from jax.experimental.pallas import tpu as pltpu
from jax.experimental.pallas import tpu_sc as plsc

N = 10000
E = 320000
B = 256
NOUT = 128
EOUT = 16
HS = 256
UOUT = 128

NC = 2
NS = 16
NW = NC * NS
L = 16

EPT = E // NW
CH = 2048
NFULL = EPT // CH
TAIL = EPT - NFULL * CH
GRPS = CH // L


def _edge_agg_sc(row, edge_attr, batch_i):
    mesh = plsc.VectorSubcoreMesh(core_axis_name="c", subcore_axis_name="s")

    @functools.partial(
        pl.kernel,
        out_type=[
            jax.ShapeDtypeStruct((NC, B, EOUT), jnp.float32),
            jax.ShapeDtypeStruct((NC, 32, 16), jnp.float32),
        ],
        mesh=mesh,
        scratch_types=[
            pltpu.VMEM((N,), jnp.int32),
            pltpu.VMEM((CH,), jnp.int32),
            pltpu.VMEM((CH, EOUT), jnp.float32),
            pltpu.VMEM((16, 128), jnp.int32),
            pltpu.VMEM((32, 16), jnp.float32),
            pltpu.VMEM((B + 16, EOUT), jnp.float32),
            pltpu.VMEM((32,), jnp.int32),
            pltpu.VMEM_SHARED((B + 16, EOUT), jnp.float32),
            pltpu.VMEM_SHARED((32, 16), jnp.float32),
        ],
    )
    def k(row_hbm, attr_hbm, batch_hbm, esum_hbm, ecnt_hbm,
          btbl, rows_v, attr_v, eb2d, cntl, zbuf, i32v,
          acc_sh, cnt_sh):
        c = lax.axis_index("c")
        s = lax.axis_index("s")
        wid = s * NC + c
        iota16 = lax.iota(jnp.int32, L)

        def zrow(i, carry):
            zbuf[i, :] = jnp.zeros((L,), jnp.float32)
            return carry
        lax.fori_loop(0, B + 16, zrow, 0)

        def crow(i, carry):
            cntl[i, :] = jnp.zeros((L,), jnp.float32)
            return carry
        lax.fori_loop(0, 32, crow, 0)

        i32v[pl.ds(0, 16)] = iota16
        i32v[pl.ds(16, 16)] = iota16 + 16

        @pl.when(s == 0)
        def _():
            pltpu.sync_copy(zbuf, acc_sh)
            pltpu.sync_copy(zbuf.at[pl.ds(0, 32), :], cnt_sh)

        pltpu.sync_copy(batch_hbm, btbl)
        plsc.subcore_barrier()

        for ci in range(NFULL + 1):
            sz = CH if ci < NFULL else TAIL
            off = wid * EPT + ci * CH
            pltpu.sync_copy(row_hbm.at[pl.ds(off, sz)], rows_v.at[pl.ds(0, sz)])
            pltpu.sync_copy(attr_hbm.at[pl.ds(off, sz), :],
                            attr_v.at[pl.ds(0, sz), :])

            last = ci == NFULL

            def grp(g, carry):
                rv = rows_v[pl.ds(g * L, L)]
                ebg = plsc.load_gather(btbl, [rv])
                if last:
                    valid = (g * L + iota16) < sz
                    eb = jnp.where(valid, ebg,
                                   jnp.full((L,), B, jnp.int32))
                else:
                    eb = ebg
                j = lax.shift_right_logical(g, 3)
                o = (g & 7) * L
                eb2d[j, pl.ds(o, L)] = eb
                rc, lastm = plsc.scan_count(eb)
                plsc.addupdate_scatter(
                    cntl,
                    [lax.shift_right_logical(eb, 4), eb & 15],
                    rc.astype(jnp.float32),
                    mask=lastm,
                )
                return carry
            lax.fori_loop(0, GRPS, grp, 0)

            for j in range(16):
                pltpu.sync_copy(attr_v.at[pl.ds(j * 128, 128), :],
                                acc_sh.at[eb2d.at[j]], add=True)

        pltpu.sync_copy(cntl, cnt_sh.at[i32v], add=True)
        plsc.subcore_barrier()

        @pl.when(s == 0)
        def _():
            pltpu.sync_copy(acc_sh.at[pl.ds(0, B), :], esum_hbm.at[c])
            pltpu.sync_copy(cnt_sh, ecnt_hbm.at[c])

    return k(row, edge_attr, batch_i)


def _node_agg_tc(x, batch_r):
    nblk = 10
    blk = N // nblk

    def body(batch_ref, x_ref, nsum_ref, ncnt_ref):
        i = pl.program_id(0)
        b = batch_ref[0]
        ohT = (lax.broadcasted_iota(jnp.int32, (B, blk), 0) == b
               ).astype(jnp.float32)
        ns = jnp.dot(ohT, x_ref[...], preferred_element_type=jnp.float32)
        nc = jnp.sum(ohT, axis=1, keepdims=True)

        @pl.when(i == 0)
        def _():
            nsum_ref[...] = jnp.zeros_like(nsum_ref)
            ncnt_ref[...] = jnp.zeros_like(ncnt_ref)

        nsum_ref[...] += ns
        ncnt_ref[...] += nc

    return pl.pallas_call(
        body,
        grid=(nblk,),
        in_specs=[
            pl.BlockSpec((1, 1, blk), lambda i: (i, 0, 0)),
            pl.BlockSpec((blk, NOUT), lambda i: (i, 0)),
        ],
        out_specs=[
            pl.BlockSpec((B, NOUT), lambda i: (0, 0)),
            pl.BlockSpec((B, 1), lambda i: (0, 0)),
        ],
        out_shape=[
            jax.ShapeDtypeStruct((B, NOUT), jnp.float32),
            jax.ShapeDtypeStruct((B, 1), jnp.float32),
        ],
    )(batch_r, x)


def _mlp_tc(u, nsum, ncnt, esum2, ecnt, w1u, w1n, w1e, b1, gamma, beta,
            w2, b2):
    def body(u_ref, ns_ref, nc_ref, es_ref, ec_ref, w1u_ref, w1n_ref,
             w1e_ref, b1_ref, g_ref, be_ref, w2_ref, b2_ref, out_ref):
        nmean = ns_ref[...] / jnp.maximum(nc_ref[...], 1.0)
        es = es_ref[0] + es_ref[1]
        emean = es / jnp.maximum(ec_ref[...], 1.0)
        h = (jnp.dot(u_ref[...], w1u_ref[...],
                     preferred_element_type=jnp.float32)
             + jnp.dot(nmean, w1n_ref[...],
                       preferred_element_type=jnp.float32)
             + jnp.dot(emean, w1e_ref[...],
                       preferred_element_type=jnp.float32)
             + b1_ref[...])
        mu = jnp.mean(h, axis=-1, keepdims=True)
        d = h - mu
        var = jnp.mean(d * d, axis=-1, keepdims=True)
        hn = d / jnp.sqrt(var + 1e-5) * g_ref[...] + be_ref[...]
        hr = jnp.maximum(hn, 0.0)
        out_ref[...] = jnp.dot(hr, w2_ref[...],
                               preferred_element_type=jnp.float32) + b2_ref[...]

    return pl.pallas_call(
        body,
        out_shape=jax.ShapeDtypeStruct((B, UOUT), jnp.float32),
    )(u, nsum, ncnt, esum2, ecnt, w1u, w1n, w1e, b1, gamma, beta, w2, b2)


def kernel(x, edge_index, edge_attr, u, batch, W1, b1, gamma, beta, W2, b2):
    row = edge_index[0].astype(jnp.int32)
    batch_i = batch.astype(jnp.int32)
    batch_r = batch_i.reshape(10, 1, N // 10)

    nsum, ncnt = _node_agg_tc(x, batch_r)
    esum_p, ecnt_p = _edge_agg_sc(row, edge_attr, batch_i)

    ecnt = (ecnt_p[0] + ecnt_p[1]).reshape(512)[:B].reshape(B, 1)

    uin = u.shape[1]
    w1u = W1[:uin]
    w1n = W1[uin:uin + NOUT]
    w1e = W1[uin + NOUT:]
    return _mlp_tc(u, nsum, ncnt, esum_p, ecnt, w1u, w1n, w1e,
                   b1.reshape(1, HS), gamma.reshape(1, HS),
                   beta.reshape(1, HS), W2, b2.reshape(1, UOUT))


# speedup vs baseline: 15.0189x; 15.0189x over previous
import functools

import jax
import jax.numpy as jnp
from jax import lax
from jax.experimental import pallas as pl
from jax.experimental.pallas import tpu as pltpu
from jax.experimental.pallas import tpu_sc as plsc

N = 10000
E = 320000
B = 256
NOUT = 128
EOUT = 16
HS = 256
UOUT = 128

NC = 2
NS = 16
NW = NC * NS
L = 16

EPT = E // NW
CH = 2048
NFULL = EPT // CH
TAIL = EPT - NFULL * CH
GRPS = CH // L


def _edge_agg_sc(row, edge_attr, batch_i):
    mesh = plsc.VectorSubcoreMesh(core_axis_name="c", subcore_axis_name="s")

    @functools.partial(
        pl.kernel,
        out_type=[
            jax.ShapeDtypeStruct((NC, B, EOUT), jnp.float32),
            jax.ShapeDtypeStruct((NC, 32, 16), jnp.float32),
        ],
        mesh=mesh,
        compiler_params=pltpu.CompilerParams(needs_layout_passes=False,
                                             use_tc_tiling_on_sc=False),
        scratch_types=[
            pltpu.VMEM((N,), jnp.int32),
            pltpu.VMEM((CH,), jnp.int32),
            pltpu.VMEM((CH, EOUT), jnp.float32),
            pltpu.VMEM((16, 128), jnp.int32),
            pltpu.VMEM((32, 16), jnp.float32),
            pltpu.VMEM((B + 16, EOUT), jnp.float32),
            pltpu.VMEM((32,), jnp.int32),
            pltpu.VMEM_SHARED((B + 16, EOUT), jnp.float32),
            pltpu.VMEM_SHARED((32, 16), jnp.float32),
        ],
    )
    def k(row_hbm, attr_hbm, batch_hbm, esum_hbm, ecnt_hbm,
          btbl, rows_v, attr_v, eb2d, cntl, zbuf, i32v,
          acc_sh, cnt_sh):
        c = lax.axis_index("c")
        s = lax.axis_index("s")
        wid = s * NC + c
        iota16 = lax.iota(jnp.int32, L)

        def zrow(i, carry):
            zbuf[i, :] = jnp.zeros((L,), jnp.float32)
            return carry
        lax.fori_loop(0, B + 16, zrow, 0)

        def crow(i, carry):
            cntl[i, :] = jnp.zeros((L,), jnp.float32)
            return carry
        lax.fori_loop(0, 32, crow, 0)

        i32v[pl.ds(0, 16)] = iota16
        i32v[pl.ds(16, 16)] = iota16 + 16

        @pl.when(s == 0)
        def _():
            pltpu.sync_copy(zbuf, acc_sh)
            pltpu.sync_copy(zbuf.at[pl.ds(0, 32), :], cnt_sh)

        pltpu.sync_copy(batch_hbm, btbl)
        plsc.subcore_barrier()

        for ci in range(NFULL + 1):
            sz = CH if ci < NFULL else TAIL
            off = wid * EPT + ci * CH
            pltpu.sync_copy(row_hbm.at[pl.ds(off, sz)], rows_v.at[pl.ds(0, sz)])
            pltpu.sync_copy(attr_hbm.at[pl.ds(off, sz), :],
                            attr_v.at[pl.ds(0, sz), :])

            last = ci == NFULL

            def grp(g, carry):
                rv = rows_v[pl.ds(g * L, L)]
                ebg = plsc.load_gather(btbl, [rv])
                if last:
                    valid = (g * L + iota16) < sz
                    eb = jnp.where(valid, ebg,
                                   jnp.full((L,), B, jnp.int32))
                else:
                    eb = ebg
                j = lax.shift_right_logical(g, 3)
                o = (g & 7) * L
                eb2d[j, pl.ds(o, L)] = eb
                rc, lastm = plsc.scan_count(eb)
                plsc.addupdate_scatter(
                    cntl,
                    [lax.shift_right_logical(eb, 4), eb & 15],
                    rc.astype(jnp.float32),
                    mask=lastm,
                )
                return carry
            lax.fori_loop(0, GRPS, grp, 0)

            for j in range(16):
                pltpu.sync_copy(attr_v.at[pl.ds(j * 128, 128), :],
                                acc_sh.at[eb2d.at[j]], add=True)

        pltpu.sync_copy(cntl, cnt_sh.at[i32v], add=True)
        plsc.subcore_barrier()

        @pl.when(s == 0)
        def _():
            pltpu.sync_copy(acc_sh.at[pl.ds(0, B), :], esum_hbm.at[c])
            pltpu.sync_copy(cnt_sh, ecnt_hbm.at[c])

    return k(row, edge_attr, batch_i)


def _node_agg_tc(x, batch_r):
    nblk = 10
    blk = N // nblk

    def body(batch_ref, x_ref, nsum_ref, ncnt_ref):
        i = pl.program_id(0)
        b = batch_ref[0]
        ohT = (lax.broadcasted_iota(jnp.int32, (B, blk), 0) == b
               ).astype(jnp.float32)
        ns = jnp.dot(ohT, x_ref[...], preferred_element_type=jnp.float32)
        nc = jnp.sum(ohT, axis=1, keepdims=True)

        @pl.when(i == 0)
        def _():
            nsum_ref[...] = jnp.zeros_like(nsum_ref)
            ncnt_ref[...] = jnp.zeros_like(ncnt_ref)

        nsum_ref[...] += ns
        ncnt_ref[...] += nc

    return pl.pallas_call(
        body,
        grid=(nblk,),
        in_specs=[
            pl.BlockSpec((1, 1, blk), lambda i: (i, 0, 0)),
            pl.BlockSpec((blk, NOUT), lambda i: (i, 0)),
        ],
        out_specs=[
            pl.BlockSpec((B, NOUT), lambda i: (0, 0)),
            pl.BlockSpec((B, 1), lambda i: (0, 0)),
        ],
        out_shape=[
            jax.ShapeDtypeStruct((B, NOUT), jnp.float32),
            jax.ShapeDtypeStruct((B, 1), jnp.float32),
        ],
    )(batch_r, x)


def _mlp_tc(u, nsum, ncnt, esum2, ecnt, w1u, w1n, w1e, b1, gamma, beta,
            w2, b2):
    def body(u_ref, ns_ref, nc_ref, es_ref, ec_ref, w1u_ref, w1n_ref,
             w1e_ref, b1_ref, g_ref, be_ref, w2_ref, b2_ref, out_ref):
        nmean = ns_ref[...] / jnp.maximum(nc_ref[...], 1.0)
        es = es_ref[0] + es_ref[1]
        emean = es / jnp.maximum(ec_ref[...], 1.0)
        h = (jnp.dot(u_ref[...], w1u_ref[...],
                     preferred_element_type=jnp.float32)
             + jnp.dot(nmean, w1n_ref[...],
                       preferred_element_type=jnp.float32)
             + jnp.dot(emean, w1e_ref[...],
                       preferred_element_type=jnp.float32)
             + b1_ref[...])
        mu = jnp.mean(h, axis=-1, keepdims=True)
        d = h - mu
        var = jnp.mean(d * d, axis=-1, keepdims=True)
        hn = d / jnp.sqrt(var + 1e-5) * g_ref[...] + be_ref[...]
        hr = jnp.maximum(hn, 0.0)
        out_ref[...] = jnp.dot(hr, w2_ref[...],
                               preferred_element_type=jnp.float32) + b2_ref[...]

    return pl.pallas_call(
        body,
        out_shape=jax.ShapeDtypeStruct((B, UOUT), jnp.float32),
    )(u, nsum, ncnt, esum2, ecnt, w1u, w1n, w1e, b1, gamma, beta, w2, b2)


def kernel(x, edge_index, edge_attr, u, batch, W1, b1, gamma, beta, W2, b2):
    row = edge_index[0].astype(jnp.int32)
    batch_i = batch.astype(jnp.int32)
    batch_r = batch_i.reshape(10, 1, N // 10)

    nsum, ncnt = _node_agg_tc(x, batch_r)
    esum_p, ecnt_p = _edge_agg_sc(row, edge_attr, batch_i)

    ecnt = (ecnt_p[0] + ecnt_p[1]).reshape(512)[:B].reshape(B, 1)

    uin = u.shape[1]
    w1u = W1[:uin]
    w1n = W1[uin:uin + NOUT]
    w1e = W1[uin + NOUT:]
    return _mlp_tc(u, nsum, ncnt, esum_p, ecnt, w1u, w1n, w1e,
                   b1.reshape(1, HS), gamma.reshape(1, HS),
                   beta.reshape(1, HS), W2, b2.reshape(1, UOUT))


# speedup vs baseline: 16.5593x; 1.1026x over previous
import functools

import jax
import jax.numpy as jnp
from jax import lax
from jax.experimental import pallas as pl
from jax.experimental.pallas import tpu as pltpu
from jax.experimental.pallas import tpu_sc as plsc

N = 10000
E = 320000
B = 256
NOUT = 128
EOUT = 16
HS = 256
UOUT = 128

NC = 2
NS = 16
NW = NC * NS
L = 16

EPT = E // NW
CH = 2048
NFULL = EPT // CH
TAIL = EPT - NFULL * CH
GRPS = CH // L


def _edge_agg_sc(ei, edge_attr, batch_i):
    mesh = plsc.VectorSubcoreMesh(core_axis_name="c", subcore_axis_name="s")

    @functools.partial(
        pl.kernel,
        out_type=[
            jax.ShapeDtypeStruct((NC, B, EOUT), jnp.float32),
            jax.ShapeDtypeStruct((NC, 32, 16), jnp.float32),
        ],
        mesh=mesh,
        compiler_params=pltpu.CompilerParams(needs_layout_passes=False,
                                             use_tc_tiling_on_sc=False),
        scratch_types=[
            pltpu.VMEM((N,), jnp.int32),
            pltpu.VMEM((CH,), jnp.int32),
            pltpu.VMEM((CH, EOUT), jnp.float32),
            pltpu.VMEM((16, 128), jnp.int32),
            pltpu.VMEM((32, 16), jnp.float32),
            pltpu.VMEM((B + 16, EOUT), jnp.float32),
            pltpu.VMEM((32,), jnp.int32),
            pltpu.VMEM_SHARED((B + 16, EOUT), jnp.float32),
            pltpu.VMEM_SHARED((32, 16), jnp.float32),
        ],
    )
    def k(ei_hbm, attr_hbm, batch_hbm, esum_hbm, ecnt_hbm,
          btbl, rows_v, attr_v, eb2d, cntl, zbuf, i32v,
          acc_sh, cnt_sh):
        c = lax.axis_index("c")
        s = lax.axis_index("s")
        wid = s * NC + c
        iota16 = lax.iota(jnp.int32, L)

        def zrow(i, carry):
            zbuf[i, :] = jnp.zeros((L,), jnp.float32)
            return carry
        lax.fori_loop(0, B + 16, zrow, 0)

        def crow(i, carry):
            cntl[i, :] = jnp.zeros((L,), jnp.float32)
            return carry
        lax.fori_loop(0, 32, crow, 0)

        i32v[pl.ds(0, 16)] = iota16
        i32v[pl.ds(16, 16)] = iota16 + 16

        @pl.when(s == 0)
        def _():
            pltpu.sync_copy(zbuf, acc_sh)
            pltpu.sync_copy(zbuf.at[pl.ds(0, 32), :], cnt_sh)

        pltpu.sync_copy(batch_hbm, btbl)
        plsc.subcore_barrier()

        for ci in range(NFULL + 1):
            sz = CH if ci < NFULL else TAIL
            off = wid * EPT + ci * CH
            pltpu.sync_copy(ei_hbm.at[0, pl.ds(off, sz)],
                            rows_v.at[pl.ds(0, sz)])
            pltpu.sync_copy(attr_hbm.at[pl.ds(off, sz), :],
                            attr_v.at[pl.ds(0, sz), :])

            last = ci == NFULL

            def grp(g, carry):
                rv = rows_v[pl.ds(g * L, L)]
                ebg = plsc.load_gather(btbl, [rv])
                if last:
                    valid = (g * L + iota16) < sz
                    eb = jnp.where(valid, ebg,
                                   jnp.full((L,), B, jnp.int32))
                else:
                    eb = ebg
                j = lax.shift_right_logical(g, 3)
                o = (g & 7) * L
                eb2d[j, pl.ds(o, L)] = eb
                rc, lastm = plsc.scan_count(eb)
                plsc.addupdate_scatter(
                    cntl,
                    [lax.shift_right_logical(eb, 4), eb & 15],
                    rc.astype(jnp.float32),
                    mask=lastm,
                )
                return carry
            lax.fori_loop(0, GRPS, grp, 0)

            for j in range(16):
                pltpu.sync_copy(attr_v.at[pl.ds(j * 128, 128), :],
                                acc_sh.at[eb2d.at[j]], add=True)

        pltpu.sync_copy(cntl, cnt_sh.at[i32v], add=True)
        plsc.subcore_barrier()

        @pl.when(s == 0)
        def _():
            pltpu.sync_copy(acc_sh.at[pl.ds(0, B), :], esum_hbm.at[c])
            pltpu.sync_copy(cnt_sh, ecnt_hbm.at[c])

    return k(ei, edge_attr, batch_i)


def _node_agg_tc(x, batch_r):
    nblk = 10
    blk = N // nblk

    def body(batch_ref, x_ref, nsum_ref, ncnt_ref):
        i = pl.program_id(0)
        b = batch_ref[0]
        ohT = (lax.broadcasted_iota(jnp.int32, (B, blk), 0) == b
               ).astype(jnp.float32)
        ns = jnp.dot(ohT, x_ref[...], preferred_element_type=jnp.float32)
        nc = jnp.sum(ohT, axis=1, keepdims=True)

        @pl.when(i == 0)
        def _():
            nsum_ref[...] = jnp.zeros_like(nsum_ref)
            ncnt_ref[...] = jnp.zeros_like(ncnt_ref)

        nsum_ref[...] += ns
        ncnt_ref[...] += nc

    return pl.pallas_call(
        body,
        grid=(nblk,),
        in_specs=[
            pl.BlockSpec((1, 1, blk), lambda i: (i, 0, 0)),
            pl.BlockSpec((blk, NOUT), lambda i: (i, 0)),
        ],
        out_specs=[
            pl.BlockSpec((B, NOUT), lambda i: (0, 0)),
            pl.BlockSpec((B, 1), lambda i: (0, 0)),
        ],
        out_shape=[
            jax.ShapeDtypeStruct((B, NOUT), jnp.float32),
            jax.ShapeDtypeStruct((B, 1), jnp.float32),
        ],
    )(batch_r, x)


def _mlp_tc(u, nsum, ncnt, esum2, ecnt, w1u, w1n, w1e, b1, gamma, beta,
            w2, b2):
    def body(u_ref, ns_ref, nc_ref, es_ref, ec_ref, w1u_ref, w1n_ref,
             w1e_ref, b1_ref, g_ref, be_ref, w2_ref, b2_ref, out_ref):
        nmean = ns_ref[...] / jnp.maximum(nc_ref[...], 1.0)
        es = es_ref[0] + es_ref[1]
        emean = es / jnp.maximum(ec_ref[...], 1.0)
        h = (jnp.dot(u_ref[...], w1u_ref[...],
                     preferred_element_type=jnp.float32)
             + jnp.dot(nmean, w1n_ref[...],
                       preferred_element_type=jnp.float32)
             + jnp.dot(emean, w1e_ref[...],
                       preferred_element_type=jnp.float32)
             + b1_ref[...])
        mu = jnp.mean(h, axis=-1, keepdims=True)
        d = h - mu
        var = jnp.mean(d * d, axis=-1, keepdims=True)
        hn = d / jnp.sqrt(var + 1e-5) * g_ref[...] + be_ref[...]
        hr = jnp.maximum(hn, 0.0)
        out_ref[...] = jnp.dot(hr, w2_ref[...],
                               preferred_element_type=jnp.float32) + b2_ref[...]

    return pl.pallas_call(
        body,
        out_shape=jax.ShapeDtypeStruct((B, UOUT), jnp.float32),
    )(u, nsum, ncnt, esum2, ecnt, w1u, w1n, w1e, b1, gamma, beta, w2, b2)


def kernel(x, edge_index, edge_attr, u, batch, W1, b1, gamma, beta, W2, b2):
    ei = edge_index.astype(jnp.int32)
    batch_i = batch.astype(jnp.int32)
    batch_r = batch_i.reshape(10, 1, N // 10)

    nsum, ncnt = _node_agg_tc(x, batch_r)
    esum_p, ecnt_p = _edge_agg_sc(ei, edge_attr, batch_i)

    ecnt = (ecnt_p[0] + ecnt_p[1]).reshape(512)[:B].reshape(B, 1)

    uin = u.shape[1]
    w1u = W1[:uin]
    w1n = W1[uin:uin + NOUT]
    w1e = W1[uin + NOUT:]
    return _mlp_tc(u, nsum, ncnt, esum_p, ecnt, w1u, w1n, w1e,
                   b1.reshape(1, HS), gamma.reshape(1, HS),
                   beta.reshape(1, HS), W2, b2.reshape(1, UOUT))
